# baseline (speedup 1.0000x reference)
import math

import jax
import jax.numpy as jnp
from jax.experimental import pallas as pl
from jax.experimental.pallas import tpu as pltpu

B, K, MEM, EDGE, TIME, OUT, H = 10000, 32, 128, 16, 16, 128, 2
DH = OUT // H
BB = 200
BK = BB * K


def _tgn_block(nm_ref, nb_ref, ef_ref, tf_ref,
               wkm_ref, wke_ref, wkt_ref,
               wvm_ref, wve_ref, wvt_ref,
               wq_ref, bq_ref, wc1_ref, wof_ref, bc_ref, mh_ref,
               out_ref):
    f32 = jnp.float32
    bf16 = jnp.bfloat16
    nb = nb_ref[...].reshape(BK, MEM).astype(bf16)
    ef = ef_ref[...].reshape(BK, EDGE).astype(bf16)
    tf = tf_ref[...].reshape(BK, TIME).astype(bf16)
    nm = nm_ref[...]
    nmh = nm.astype(bf16)

    kp = (jnp.dot(nb, wkm_ref[...], preferred_element_type=f32)
          + jnp.dot(ef, wke_ref[...], preferred_element_type=f32)
          + jnp.dot(tf, wkt_ref[...], preferred_element_type=f32))
    vp = (jnp.dot(nb, wvm_ref[...], preferred_element_type=f32)
          + jnp.dot(ef, wve_ref[...], preferred_element_type=f32)
          + jnp.dot(tf, wvt_ref[...], preferred_element_type=f32))
    qp = jnp.dot(nmh, wq_ref[...], preferred_element_type=f32) + bq_ref[...]

    prod = (kp.reshape(BB, K, OUT) * qp.reshape(BB, 1, OUT)).reshape(BK, OUT)
    s = jnp.dot(prod.astype(bf16), mh_ref[...], preferred_element_type=f32)
    e3 = jnp.exp(s).reshape(BB, K, OUT)
    vp3 = vp.reshape(BB, K, OUT)
    ctx_un = jnp.sum(e3 * vp3, axis=1)
    denom = jnp.sum(e3, axis=1)
    ctx = ctx_un / denom

    z = (jnp.dot(nmh, wc1_ref[...], preferred_element_type=f32)
         + jnp.dot(ctx.astype(bf16), wof_ref[...], preferred_element_type=f32)
         + bc_ref[...])
    out_ref[...] = jnp.maximum(z, 0.0)


@jax.jit
def kernel(node_mems, neigh_mems, neigh_edge_feats, neigh_dt_enc,
           W_key, b_key, W_val, b_val, W_query, b_query,
           W_q, b_q, W_k, b_k, W_v, b_v, W_o, b_o, W_comb, b_comb):
    scale = 1.0 / math.sqrt(DH)
    Wk_f = (W_k @ W_key) * scale
    Wv_f = W_v @ W_val
    bv_f = W_v @ b_val + b_v
    Wq_f = W_q @ W_query
    bq_f = W_q @ b_query + b_q
    Wc1 = W_comb[:, :MEM]
    Wc2 = W_comb[:, MEM:]
    Wo_f = Wc2 @ W_o
    bc_f = b_comb + Wc2 @ b_o + Wo_f @ bv_f

    bf16 = jnp.bfloat16
    wkm = Wk_f[:, :MEM].T.astype(bf16)
    wke = Wk_f[:, MEM:MEM + EDGE].T.astype(bf16)
    wkt = Wk_f[:, MEM + EDGE:].T.astype(bf16)
    wvm = Wv_f[:, :MEM].T.astype(bf16)
    wve = Wv_f[:, MEM:MEM + EDGE].T.astype(bf16)
    wvt = Wv_f[:, MEM + EDGE:].T.astype(bf16)
    wq = Wq_f.T.astype(bf16)
    wc1 = Wc1.T.astype(bf16)
    wof = Wo_f.T.astype(bf16)

    lane = jnp.arange(OUT)
    mh = (lane[:, None] // DH == lane[None, :] // DH).astype(bf16)

    def row2d(v):
        return v.reshape(1, OUT)

    grid = (B // BB,)
    full = lambda shape: pl.BlockSpec(shape, lambda i: tuple(0 for _ in shape))
    out = pl.pallas_call(
        _tgn_block,
        grid=grid,
        in_specs=[
            pl.BlockSpec((BB, MEM), lambda i: (i, 0)),
            pl.BlockSpec((BB, K, MEM), lambda i: (i, 0, 0)),
            pl.BlockSpec((BB, K, EDGE), lambda i: (i, 0, 0)),
            pl.BlockSpec((BB, K, TIME), lambda i: (i, 0, 0)),
            full((MEM, OUT)), full((EDGE, OUT)), full((TIME, OUT)),
            full((MEM, OUT)), full((EDGE, OUT)), full((TIME, OUT)),
            full((MEM, OUT)), full((1, OUT)),
            full((MEM, OUT)), full((OUT, OUT)), full((1, OUT)),
            full((OUT, OUT)),
        ],
        out_specs=pl.BlockSpec((BB, OUT), lambda i: (i, 0)),
        out_shape=jax.ShapeDtypeStruct((B, OUT), jnp.float32),
        compiler_params=pltpu.CompilerParams(
            dimension_semantics=("arbitrary",),
        ),
    )(node_mems, neigh_mems, neigh_edge_feats, neigh_dt_enc,
      wkm, wke, wkt,
      wvm, wve, wvt,
      wq, row2d(bq_f),
      wc1, wof, row2d(bc_f), mh)
    return out


# speedup vs baseline: 1.2449x; 1.2449x over previous
import math

import jax
import jax.numpy as jnp
from jax.experimental import pallas as pl
from jax.experimental.pallas import tpu as pltpu

B, K, MEM, EDGE, TIME, OUT, H = 10000, 32, 128, 16, 16, 128, 2
DH = OUT // H
BB = 200
BK = BB * K


def _tgn_block(nm_ref, nb_ref, ef_ref, tf_ref,
               wkm_ref, wke_ref, wkt_ref,
               wvm_ref, wve_ref, wvt_ref,
               wq_ref, bq_ref, wc1_ref, wof_ref, bc_ref, mh_ref,
               out_ref):
    f32 = jnp.float32
    bf16 = jnp.bfloat16
    nb = nb_ref[...].astype(bf16)
    ef = ef_ref[...].astype(bf16)
    tf = tf_ref[...].astype(bf16)
    nm = nm_ref[...]
    nmh = nm.astype(bf16)

    kp = (jnp.dot(nb, wkm_ref[...], preferred_element_type=f32)
          + jnp.dot(ef, wke_ref[...], preferred_element_type=f32)
          + jnp.dot(tf, wkt_ref[...], preferred_element_type=f32))
    vp = (jnp.dot(nb, wvm_ref[...], preferred_element_type=f32)
          + jnp.dot(ef, wve_ref[...], preferred_element_type=f32)
          + jnp.dot(tf, wvt_ref[...], preferred_element_type=f32))
    qp = jnp.dot(nmh, wq_ref[...], preferred_element_type=f32) + bq_ref[...]

    prod = (kp.reshape(BB, K, OUT) * qp.reshape(BB, 1, OUT)).reshape(BK, OUT)
    s = jnp.dot(prod.astype(bf16), mh_ref[...], preferred_element_type=f32)
    e3 = jnp.exp(s).reshape(BB, K, OUT)
    vp3 = vp.reshape(BB, K, OUT)
    ctx_un = jnp.sum(e3 * vp3, axis=1)
    denom = jnp.sum(e3, axis=1)
    ctx = ctx_un / denom

    z = (jnp.dot(nmh, wc1_ref[...], preferred_element_type=f32)
         + jnp.dot(ctx.astype(bf16), wof_ref[...], preferred_element_type=f32)
         + bc_ref[...])
    out_ref[...] = jnp.maximum(z, 0.0)


@jax.jit
def kernel(node_mems, neigh_mems, neigh_edge_feats, neigh_dt_enc,
           W_key, b_key, W_val, b_val, W_query, b_query,
           W_q, b_q, W_k, b_k, W_v, b_v, W_o, b_o, W_comb, b_comb):
    scale = 1.0 / math.sqrt(DH)
    Wk_f = (W_k @ W_key) * scale
    Wv_f = W_v @ W_val
    bv_f = W_v @ b_val + b_v
    Wq_f = W_q @ W_query
    bq_f = W_q @ b_query + b_q
    Wc1 = W_comb[:, :MEM]
    Wc2 = W_comb[:, MEM:]
    Wo_f = Wc2 @ W_o
    bc_f = b_comb + Wc2 @ b_o + Wo_f @ bv_f

    bf16 = jnp.bfloat16
    wkm = Wk_f[:, :MEM].T.astype(bf16)
    wke = Wk_f[:, MEM:MEM + EDGE].T.astype(bf16)
    wkt = Wk_f[:, MEM + EDGE:].T.astype(bf16)
    wvm = Wv_f[:, :MEM].T.astype(bf16)
    wve = Wv_f[:, MEM:MEM + EDGE].T.astype(bf16)
    wvt = Wv_f[:, MEM + EDGE:].T.astype(bf16)
    wq = Wq_f.T.astype(bf16)
    wc1 = Wc1.T.astype(bf16)
    wof = Wo_f.T.astype(bf16)

    lane = jnp.arange(OUT)
    mh = (lane[:, None] // DH == lane[None, :] // DH).astype(bf16)

    def row2d(v):
        return v.reshape(1, OUT)

    grid = (B // BB,)
    full = lambda shape: pl.BlockSpec(shape, lambda i: tuple(0 for _ in shape))
    out = pl.pallas_call(
        _tgn_block,
        grid=grid,
        in_specs=[
            pl.BlockSpec((BB, MEM), lambda i: (i, 0)),
            pl.BlockSpec((BK, MEM), lambda i: (i, 0)),
            pl.BlockSpec((BK, EDGE), lambda i: (i, 0)),
            pl.BlockSpec((BK, TIME), lambda i: (i, 0)),
            full((MEM, OUT)), full((EDGE, OUT)), full((TIME, OUT)),
            full((MEM, OUT)), full((EDGE, OUT)), full((TIME, OUT)),
            full((MEM, OUT)), full((1, OUT)),
            full((MEM, OUT)), full((OUT, OUT)), full((1, OUT)),
            full((OUT, OUT)),
        ],
        out_specs=pl.BlockSpec((BB, OUT), lambda i: (i, 0)),
        out_shape=jax.ShapeDtypeStruct((B, OUT), jnp.float32),
        compiler_params=pltpu.CompilerParams(
            dimension_semantics=("arbitrary",),
        ),
    )(node_mems,
      neigh_mems.reshape(B * K, MEM),
      neigh_edge_feats.reshape(B * K, EDGE),
      neigh_dt_enc.reshape(B * K, TIME),
      wkm, wke, wkt,
      wvm, wve, wvt,
      wq, row2d(bq_f),
      wc1, wof, row2d(bc_f), mh)
    return out
